# baseline (speedup 1.0000x reference)
import functools
import math

import jax
import jax.numpy as jnp
from jax import lax
from jax.experimental import pallas as pl
from jax.experimental.pallas import tpu as pltpu
from jax.experimental.pallas import tpu_sc as plsc

N = 50000
E = 800000
EPS = 1e-5

NC, NS, L = 2, 16, 16
NP = 51200
EP = 802816
B = 64
F = 32
RPT = NP // NS
ZROWS = 128
EPW = EP // NS
NBLK = EPW // B
EPW2 = EP // (NS * NC)
NBLK2 = EPW2 // B

RBLK = 512
NGRID = NP // RBLK

C_SELF = math.tanh(2.0) * 0.5 + 0.6
BN_INV = 1.0 / math.sqrt(1.0 + EPS)

_MESH = plsc.VectorSubcoreMesh(
    core_axis_name="c", subcore_axis_name="s", num_cores=NC, num_subcores=NS)


SB2 = 4


def _hist_call(col2d, ones_in, zeros_in):
    @functools.partial(
        pl.kernel,
        out_type=jax.ShapeDtypeStruct((NC, NP, 8), jnp.float32),
        mesh=_MESH,
        compiler_params=pltpu.CompilerParams(use_tc_tiling_on_sc=False),
        scratch_types=[
            pltpu.VMEM((SB2, B), jnp.int32),
            pltpu.VMEM((B, 8), jnp.float32),
            pltpu.VMEM_SHARED((NP, 8), jnp.float32),
        ],
    )
    def hist_kernel(col_hbm, ones_hbm, zeros_hbm, out_hbm, col_s, ones_v, acc):
        cid = lax.axis_index("c")
        sid = lax.axis_index("s")
        wid = sid * NC + cid
        pltpu.sync_copy(ones_hbm, ones_v)
        pltpu.sync_copy(zeros_hbm, acc.at[pl.ds(sid * RPT, RPT)])
        plsc.subcore_barrier()

        def sb_body(s, _):
            srow = wid * (NBLK2 // SB2) * SB2 + s * SB2
            pltpu.sync_copy(col_hbm.at[pl.ds(srow, SB2)], col_s)
            for b in range(SB2):
                pltpu.sync_copy(ones_v, acc.at[col_s.at[b]], add=True)
            return 0

        lax.fori_loop(0, NBLK2 // SB2, sb_body, 0)
        plsc.subcore_barrier()
        pltpu.sync_copy(acc.at[pl.ds(sid * RPT, RPT)],
                        out_hbm.at[cid, pl.ds(sid * RPT, RPT)])

    return hist_kernel(col2d, ones_in, zeros_in)


SB = 8


def _make_layer_kernel(C):
    chunks0 = list(range(0, (C + 1) // 2))
    chunks1 = list(range((C + 1) // 2, C))

    NSB = NBLK // SB

    @functools.partial(
        pl.kernel,
        out_type=jax.ShapeDtypeStruct((C, NP, F), jnp.float32),
        mesh=_MESH,
        compiler_params=pltpu.CompilerParams(use_tc_tiling_on_sc=False),
        scratch_types=[
            pltpu.VMEM((2, SB, B), jnp.int32),
            pltpu.VMEM((2, SB, B), jnp.int32),
            pltpu.VMEM((2, SB, B), jnp.float32),
            pltpu.VMEM((B, F), jnp.float32),
            pltpu.VMEM((B, F), jnp.float32),
            pltpu.VMEM((B, F), jnp.float32),
            pltpu.VMEM((B, F), jnp.float32),
            pltpu.VMEM((ZROWS, F), jnp.float32),
            pltpu.VMEM_SHARED((NP, F), jnp.float32),
            [pltpu.SemaphoreType.DMA] * 2,
            [pltpu.SemaphoreType.DMA] * 2,
            [pltpu.SemaphoreType.DMA] * 2,
        ],
    )
    def layer_kernel(hp_hbm, row_hbm, col_hbm, ew_hbm, z_hbm, out_hbm,
                     row_s, col_s, ew_s, rows0, rows1, srows0, srows1,
                     zbuf, acc, gsems, ssems, isems):
        cid = lax.axis_index("c")
        sid = lax.axis_index("s")
        pltpu.sync_copy(z_hbm, zbuf)

        def idx_load(p, s_dyn):
            srow = sid * NBLK + s_dyn * SB
            pltpu.async_copy(row_hbm.at[pl.ds(srow, SB)], row_s.at[p],
                             isems[p])
            pltpu.async_copy(col_hbm.at[pl.ds(srow, SB)], col_s.at[p],
                             isems[p])
            pltpu.async_copy(ew_hbm.at[pl.ds(srow, SB)], ew_s.at[p], isems[p])

        def idx_wait(p):
            for src, ref in ((row_hbm, row_s), (col_hbm, col_s),
                             (ew_hbm, ew_s)):
                pltpu.make_async_copy(
                    src.at[pl.ds(0, SB)], ref.at[p], isems[p]).wait()

        grefs = [rows0, rows1]
        srefs = [srows0, srows1]

        def gather(c, p, b, r):
            pltpu.async_copy(
                hp_hbm.at[c].at[row_s.at[p].at[b]], grefs[r], gsems[r])

        def gather_wait(c, r):
            pltpu.make_async_copy(
                hp_hbm.at[c].at[row_s.at[0].at[0]], grefs[r],
                gsems[r]).wait()

        def scatter(c, p, b, r):
            pltpu.async_copy(
                srefs[r], acc.at[col_s.at[p].at[b]], ssems[r], add=True)

        def scatter_wait(c, r):
            pltpu.make_async_copy(
                srefs[r], acc.at[col_s.at[0].at[0]], ssems[r]).wait()

        def do_chunk(c):
            for k in range(RPT // ZROWS):
                pltpu.sync_copy(
                    zbuf, acc.at[pl.ds(sid * RPT + k * ZROWS, ZROWS)])
            idx_load(0, 0)
            idx_wait(0)
            gather(c, 0, 0, 0)
            plsc.subcore_barrier()

            def sb_iter(s, cur, nxt):
                @pl.when(s < NSB - 1)
                def _():
                    idx_load(nxt, s + 1)
                for b in range(SB):
                    r = b % 2
                    rg, rs = grefs[r], srefs[r]
                    if b < SB - 1:
                        if b == SB - 2:
                            @pl.when(s < NSB - 1)
                            def _():
                                idx_wait(nxt)
                        gather(c, cur, b + 1, 1 - r)
                    else:
                        @pl.when(s < NSB - 1)
                        def _():
                            gather(c, nxt, 0, 1 - r)
                    gather_wait(c, r)
                    if b >= 2:
                        scatter_wait(c, r)
                    else:
                        @pl.when(s > 0)
                        def _():
                            scatter_wait(c, r)

                    def scale_body(g, _):
                        off = pl.multiple_of(g * L, L)
                        w16 = ew_s[cur, b, pl.ds(off, L)]
                        for k in range(L):
                            e = off + k
                            w = w16[k]
                            rs[e, pl.ds(0, L)] = rg[e, pl.ds(0, L)] * w
                            rs[e, pl.ds(L, L)] = rg[e, pl.ds(L, L)] * w
                        return 0

                    if True:
                        pass
                    else:
                        lax.fori_loop(0, B // L, scale_body, 0)
                    scatter(c, cur, b, r)

            def sb_body(s, _):
                @pl.when(s % 2 == 0)
                def _():
                    sb_iter(s, 0, 1)

                @pl.when(s % 2 == 1)
                def _():
                    sb_iter(s, 1, 0)
                return 0

            lax.fori_loop(0, NSB, sb_body, 0)
            scatter_wait(c, 0)
            scatter_wait(c, 1)
            plsc.subcore_barrier()
            pltpu.sync_copy(acc.at[pl.ds(sid * RPT, RPT)],
                            out_hbm.at[c, pl.ds(sid * RPT, RPT)])
            plsc.subcore_barrier()

        @pl.when(cid == 0)
        def _():
            for c in chunks0:
                do_chunk(c)

        @pl.when(cid == 1)
        def _():
            for c in chunks1:
                do_chunk(c)

    return layer_kernel


def _ew_kernel(ea_ref, out_ref):
    i = pl.program_id(0)
    r = lax.broadcasted_iota(jnp.int32, (B, 128), 0)
    c = lax.broadcasted_iota(jnp.int32, (B, 128), 1)
    flat = (i * B + r) * 128 + c
    a = ea_ref[...]
    w = jnp.tanh(a * 2.0) * 0.5 + 0.6
    out_ref[...] = jnp.where(flat < E, w, 0.0)


def _pre_kernel(x_ref, hist_ref, w1t_ref, hp_ref, dis_ref):
    deg = hist_ref[0] + hist_ref[1] + 1.0
    dis = lax.rsqrt(deg)
    a = jnp.dot(x_ref[...], w1t_ref[...], preferred_element_type=jnp.float32)
    hp = dis[:, 0:1] * a
    for c in range(hp_ref.shape[0]):
        hp_ref[c] = hp[:, c * F:(c + 1) * F]
    dis_ref[...] = dis


def _mid_kernel(C, Cn, s_ref, hp_ref, dis_ref, bgt_ref, wt_ref, hpn_ref):
    sf = jnp.concatenate([s_ref[c] for c in range(C)], axis=1)
    hf = jnp.concatenate([hp_ref[c] for c in range(C)], axis=1)
    z = dis_ref[:, 0:1] * (sf + C_SELF * hf) + bgt_ref[0:1, :]
    y = jnp.maximum(bgt_ref[1:2, :] * (z * BN_INV) + bgt_ref[2:3, :], 0.0)
    an = jnp.dot(y, wt_ref[...], preferred_element_type=jnp.float32)
    hp = dis_ref[:, 0:1] * an
    for c in range(Cn):
        hpn_ref[c] = hp[:, c * F:(c + 1) * F]


def _final_kernel(s_ref, hp_ref, dis_ref, b_ref, out_ref):
    out_ref[...] = (dis_ref[:, 0:1] * (s_ref[0] + C_SELF * hp_ref[0])
                    + b_ref[0:1, :])


def _row_spec(d):
    return pl.BlockSpec((RBLK, d), lambda i: (i, 0))


def _chunk_spec(C):
    return pl.BlockSpec((C, RBLK, F), lambda i: (0, i, 0))


def _full_spec(shape):
    return pl.BlockSpec(shape, lambda i: tuple(0 for _ in shape))


def _tc_pre(x_pad, hist, w1t):
    return pl.pallas_call(
        _pre_kernel,
        grid=(NGRID,),
        in_specs=[
            _row_spec(3),
            pl.BlockSpec((NC, RBLK, 8), lambda i: (0, i, 0)),
            _full_spec((3, 128)),
        ],
        out_specs=[_chunk_spec(4), _row_spec(8)],
        out_shape=[
            jax.ShapeDtypeStruct((4, NP, F), jnp.float32),
            jax.ShapeDtypeStruct((NP, 8), jnp.float32),
        ],
    )(x_pad, hist, w1t)


def _tc_mid(C, Cn, d, dn, s, hp, dis, bgt, wt):
    return pl.pallas_call(
        functools.partial(_mid_kernel, C, Cn),
        grid=(NGRID,),
        in_specs=[
            _chunk_spec(C),
            _chunk_spec(C),
            _row_spec(8),
            _full_spec((3, d)),
            _full_spec((d, dn)),
        ],
        out_specs=_chunk_spec(Cn),
        out_shape=jax.ShapeDtypeStruct((Cn, NP, F), jnp.float32),
    )(s, hp, dis, bgt, wt)


def _tc_final(s, hp, dis, b7p):
    return pl.pallas_call(
        _final_kernel,
        grid=(NGRID,),
        in_specs=[
            _chunk_spec(1),
            _chunk_spec(1),
            _row_spec(8),
            _full_spec((1, F)),
        ],
        out_specs=_row_spec(F),
        out_shape=jax.ShapeDtypeStruct((NP, F), jnp.float32),
    )(s, hp, dis, b7p)


def _tc_ew(ea2d):
    return pl.pallas_call(
        _ew_kernel,
        grid=(EP // 128 // B,),
        in_specs=[pl.BlockSpec((B, 128), lambda i: (i, 0))],
        out_specs=pl.BlockSpec((B, 128), lambda i: (i, 0)),
        out_shape=jax.ShapeDtypeStruct((EP // 128, 128), jnp.float32),
    )(ea2d)


def kernel(x, edge_index, edge_attr, W1, W2, W3, W4, W5, W6, W7,
           b1, b2, b3, b4, b5, b6, b7, g1, g2, g3, g4, g5, g6,
           t1, t2, t3, t4, t5, t6):
    f32 = jnp.float32
    pad_e = EP - E
    row_pad = jnp.concatenate(
        [edge_index[0], jnp.zeros((pad_e,), jnp.int32)])
    col_pad = jnp.concatenate(
        [edge_index[1], jnp.full((pad_e,), N, jnp.int32)])
    ea_pad = jnp.concatenate([edge_attr[:, 0], jnp.zeros((pad_e,), f32)])
    x_pad = jnp.concatenate([x, jnp.zeros((NP - N, 3), f32)], axis=0)

    ones_in = jnp.ones((B, 8), f32)
    zeros_h = jnp.zeros((RPT, 8), f32)
    zeros_l = jnp.zeros((ZROWS, F), f32)

    row_pad = (jnp.arange(EP, dtype=jnp.int32) % N)
    col_pad = (jnp.arange(EP, dtype=jnp.int32) % N)
    row2d = row_pad.reshape(EP // B, B)
    col2d = col_pad.reshape(EP // B, B)
    ew2d = _tc_ew(ea_pad.reshape(EP // 128, 128)).reshape(EP // B, B)
    hist = _hist_call(col2d, ones_in, zeros_h)

    w1t = W1.T
    hp, dis = _tc_pre(x_pad, hist, w1t)

    lk4 = _make_layer_kernel(4)
    lk2 = _make_layer_kernel(2)
    lk1 = _make_layer_kernel(1)

    Ws = [W2, W3, W4, W5, W6]
    bgts = [jnp.stack([b, g, t]) for b, g, t in
            [(b1, g1, t1), (b2, g2, t2), (b3, g3, t3),
             (b4, g4, t4), (b5, g5, t5), (b6, g6, t6)]]
    dims = [128, 128, 128, 64, 64, 64]
    for li in range(6):
        d = dims[li]
        C = d // F
        lk = {4: lk4, 2: lk2, 1: lk1}[C]
        s = lk(hp, row2d, col2d, ew2d, zeros_l)
        if li < 5:
            dn = dims[li + 1]
            wt = Ws[li].T
        else:
            dn = F
            wt = jnp.concatenate(
                [W7.T, jnp.zeros((64, F - 1), f32)], axis=1)
        Cn = dn // F
        hp = _tc_mid(C, Cn, d, dn, s, hp, dis, bgts[li], wt)

    s7 = lk1(hp, row2d, col2d, ew2d, zeros_l)
    b7p = jnp.concatenate([b7, jnp.zeros((F - 1,), f32)]).reshape(1, F)
    out_full = _tc_final(s7, hp, dis, b7p)
    return out_full[:N, 0:1]


# speedup vs baseline: 1.1442x; 1.1442x over previous
import functools
import math

import jax
import jax.numpy as jnp
from jax import lax
from jax.experimental import pallas as pl
from jax.experimental.pallas import tpu as pltpu
from jax.experimental.pallas import tpu_sc as plsc

N = 50000
E = 800000
EPS = 1e-5

NC, NS, L = 2, 16, 16
NP = 51200
EP = 802816
B = 128
F = 32
RPT = NP // NS
ZROWS = 128
EPW = EP // NS
NBLK = EPW // B
EPW2 = EP // (NS * NC)
NBLK2 = EPW2 // B

RBLK = 512
NGRID = NP // RBLK

C_SELF = math.tanh(2.0) * 0.5 + 0.6
BN_INV = 1.0 / math.sqrt(1.0 + EPS)

_MESH = plsc.VectorSubcoreMesh(
    core_axis_name="c", subcore_axis_name="s", num_cores=NC, num_subcores=NS)


SB2 = 4


def _hist_call(col2d, ones_in, zeros_in):
    @functools.partial(
        pl.kernel,
        out_type=jax.ShapeDtypeStruct((NC, NP, 8), jnp.float32),
        mesh=_MESH,
        compiler_params=pltpu.CompilerParams(use_tc_tiling_on_sc=False),
        scratch_types=[
            pltpu.VMEM((SB2, B), jnp.int32),
            pltpu.VMEM((B, 8), jnp.float32),
            pltpu.VMEM_SHARED((NP, 8), jnp.float32),
        ],
    )
    def hist_kernel(col_hbm, ones_hbm, zeros_hbm, out_hbm, col_s, ones_v, acc):
        cid = lax.axis_index("c")
        sid = lax.axis_index("s")
        wid = sid * NC + cid
        pltpu.sync_copy(ones_hbm, ones_v)
        pltpu.sync_copy(zeros_hbm, acc.at[pl.ds(sid * RPT, RPT)])
        plsc.subcore_barrier()

        def sb_body(s, _):
            srow = wid * (NBLK2 // SB2) * SB2 + s * SB2
            pltpu.sync_copy(col_hbm.at[pl.ds(srow, SB2)], col_s)
            for b in range(SB2):
                pltpu.sync_copy(ones_v, acc.at[col_s.at[b]], add=True)
            return 0

        lax.fori_loop(0, NBLK2 // SB2, sb_body, 0)
        plsc.subcore_barrier()
        pltpu.sync_copy(acc.at[pl.ds(sid * RPT, RPT)],
                        out_hbm.at[cid, pl.ds(sid * RPT, RPT)])

    return hist_kernel(col2d, ones_in, zeros_in)


SB = 8


def _make_layer_kernel(C, edge_split=False):
    chunks0 = list(range(0, (C + 1) // 2))
    chunks1 = list(range((C + 1) // 2, C))

    SBv = 7 if edge_split else SB
    nblk = NBLK // 2 if edge_split else NBLK
    NSB = nblk // SBv
    assert NSB * SBv == nblk
    n_out = 2 if edge_split else C

    @functools.partial(
        pl.kernel,
        out_type=jax.ShapeDtypeStruct((n_out, NP, F), jnp.float32),
        mesh=_MESH,
        compiler_params=pltpu.CompilerParams(use_tc_tiling_on_sc=False),
        scratch_types=[
            pltpu.VMEM((2, SBv, B), jnp.int32),
            pltpu.VMEM((2, SBv, B), jnp.int32),
            pltpu.VMEM((2, SBv, B), jnp.float32),
            pltpu.VMEM((B, F), jnp.float32),
            pltpu.VMEM((B, F), jnp.float32),
            pltpu.VMEM((B, F), jnp.float32),
            pltpu.VMEM((B, F), jnp.float32),
            pltpu.VMEM((ZROWS, F), jnp.float32),
            pltpu.VMEM_SHARED((NP, F), jnp.float32),
            [pltpu.SemaphoreType.DMA] * 2,
            [pltpu.SemaphoreType.DMA] * 2,
            [pltpu.SemaphoreType.DMA] * 2,
        ],
    )
    def layer_kernel(hp_hbm, row_hbm, col_hbm, ew_hbm, z_hbm, out_hbm,
                     row_s, col_s, ew_s, rows0, rows1, srows0, srows1,
                     zbuf, acc, gsems, ssems, isems):
        cid = lax.axis_index("c")
        sid = lax.axis_index("s")
        pltpu.sync_copy(z_hbm, zbuf)
        if edge_split:
            tile_base = (cid * NS + sid) * nblk
        else:
            tile_base = sid * nblk

        def idx_load(p, s_dyn):
            srow = tile_base + s_dyn * SBv
            pltpu.async_copy(row_hbm.at[pl.ds(srow, SBv)], row_s.at[p],
                             isems[p])
            pltpu.async_copy(col_hbm.at[pl.ds(srow, SBv)], col_s.at[p],
                             isems[p])
            pltpu.async_copy(ew_hbm.at[pl.ds(srow, SBv)], ew_s.at[p],
                             isems[p])

        def idx_wait(p):
            for src, ref in ((row_hbm, row_s), (col_hbm, col_s),
                             (ew_hbm, ew_s)):
                pltpu.make_async_copy(
                    src.at[pl.ds(0, SBv)], ref.at[p], isems[p]).wait()

        grefs = [rows0, rows1]
        srefs = [srows0, srows1]

        def gather(c, p, b, r):
            pltpu.async_copy(
                hp_hbm.at[c].at[row_s.at[p].at[b]], grefs[r], gsems[r])

        def gather_wait(c, r):
            pltpu.make_async_copy(
                hp_hbm.at[c].at[row_s.at[0].at[0]], grefs[r],
                gsems[r]).wait()

        def scatter(c, p, b, r):
            pltpu.async_copy(
                srefs[r], acc.at[col_s.at[p].at[b]], ssems[r], add=True)

        def scatter_wait(c, r):
            pltpu.make_async_copy(
                srefs[r], acc.at[col_s.at[0].at[0]], ssems[r]).wait()

        def do_chunk(c, oc):
            for k in range(RPT // ZROWS):
                pltpu.sync_copy(
                    zbuf, acc.at[pl.ds(sid * RPT + k * ZROWS, ZROWS)])
            idx_load(0, 0)
            idx_wait(0)
            gather(c, 0, 0, 0)
            plsc.subcore_barrier()

            def sb_iter(s, cur, nxt):
                @pl.when(s < NSB - 1)
                def _():
                    idx_load(nxt, s + 1)
                for b in range(SBv):
                    r = (b + (SBv % 2) * cur) % 2
                    rg, rs = grefs[r], srefs[r]
                    if b < SBv - 1:
                        if b == SBv - 2:
                            @pl.when(s < NSB - 1)
                            def _():
                                idx_wait(nxt)
                        gather(c, cur, b + 1, 1 - r)
                    else:
                        @pl.when(s < NSB - 1)
                        def _():
                            gather(c, nxt, 0, 1 - r)
                    gather_wait(c, r)
                    if b >= 2:
                        scatter_wait(c, r)
                    else:
                        @pl.when(s > 0)
                        def _():
                            scatter_wait(c, r)

                    def scale_body(g, _):
                        off = pl.multiple_of(g * L, L)
                        w16 = ew_s[cur, b, pl.ds(off, L)]
                        for k in range(L):
                            e = off + k
                            w = w16[k]
                            rs[e, pl.ds(0, L)] = rg[e, pl.ds(0, L)] * w
                            rs[e, pl.ds(L, L)] = rg[e, pl.ds(L, L)] * w
                        return 0

                    lax.fori_loop(0, B // L, scale_body, 0)
                    scatter(c, cur, b, r)

            def sb_body(s, _):
                @pl.when(s % 2 == 0)
                def _():
                    sb_iter(s, 0, 1)

                @pl.when(s % 2 == 1)
                def _():
                    sb_iter(s, 1, 0)
                return 0

            lax.fori_loop(0, NSB, sb_body, 0)
            scatter_wait(c, 0)
            scatter_wait(c, 1)
            plsc.subcore_barrier()
            pltpu.sync_copy(acc.at[pl.ds(sid * RPT, RPT)],
                            out_hbm.at[oc, pl.ds(sid * RPT, RPT)])
            plsc.subcore_barrier()

        if edge_split:
            do_chunk(0, cid)
        else:
            @pl.when(cid == 0)
            def _():
                for c in chunks0:
                    do_chunk(c, c)

            @pl.when(cid == 1)
            def _():
                for c in chunks1:
                    do_chunk(c, c)

    return layer_kernel


def _ew_kernel(ea_ref, out_ref):
    i = pl.program_id(0)
    r = lax.broadcasted_iota(jnp.int32, (B, 128), 0)
    c = lax.broadcasted_iota(jnp.int32, (B, 128), 1)
    flat = (i * B + r) * 128 + c
    a = ea_ref[...]
    w = jnp.tanh(a * 2.0) * 0.5 + 0.6
    out_ref[...] = jnp.where(flat < E, w, 0.0)


def _pre_kernel(x_ref, hist_ref, w1t_ref, hp_ref, dis_ref):
    deg = hist_ref[0] + hist_ref[1] + 1.0
    dis = lax.rsqrt(deg)
    a = jnp.dot(x_ref[...], w1t_ref[...], preferred_element_type=jnp.float32)
    hp = dis[:, 0:1] * a
    for c in range(hp_ref.shape[0]):
        hp_ref[c] = hp[:, c * F:(c + 1) * F]
    dis_ref[...] = dis


def _mid_kernel(C, Cn, s_ref, hp_ref, dis_ref, bgt_ref, wt_ref, hpn_ref):
    sf = jnp.concatenate([s_ref[c] for c in range(C)], axis=1)
    hf = jnp.concatenate([hp_ref[c] for c in range(C)], axis=1)
    z = dis_ref[:, 0:1] * (sf + C_SELF * hf) + bgt_ref[0:1, :]
    y = jnp.maximum(bgt_ref[1:2, :] * (z * BN_INV) + bgt_ref[2:3, :], 0.0)
    an = jnp.dot(y, wt_ref[...], preferred_element_type=jnp.float32)
    hp = dis_ref[:, 0:1] * an
    for c in range(Cn):
        hpn_ref[c] = hp[:, c * F:(c + 1) * F]


def _final_kernel(s_ref, hp_ref, dis_ref, b_ref, out_ref):
    sf = s_ref[0] + s_ref[1]
    out_ref[...] = (dis_ref[:, 0:1] * (sf + C_SELF * hp_ref[0])
                    + b_ref[0:1, :])


def _row_spec(d):
    return pl.BlockSpec((RBLK, d), lambda i: (i, 0))


def _chunk_spec(C):
    return pl.BlockSpec((C, RBLK, F), lambda i: (0, i, 0))


def _full_spec(shape):
    return pl.BlockSpec(shape, lambda i: tuple(0 for _ in shape))


def _tc_pre(x_pad, hist, w1t):
    return pl.pallas_call(
        _pre_kernel,
        grid=(NGRID,),
        in_specs=[
            _row_spec(3),
            pl.BlockSpec((NC, RBLK, 8), lambda i: (0, i, 0)),
            _full_spec((3, 128)),
        ],
        out_specs=[_chunk_spec(4), _row_spec(8)],
        out_shape=[
            jax.ShapeDtypeStruct((4, NP, F), jnp.float32),
            jax.ShapeDtypeStruct((NP, 8), jnp.float32),
        ],
    )(x_pad, hist, w1t)


def _tc_mid(C, Cn, d, dn, s, hp, dis, bgt, wt):
    return pl.pallas_call(
        functools.partial(_mid_kernel, C, Cn),
        grid=(NGRID,),
        in_specs=[
            _chunk_spec(C),
            _chunk_spec(C),
            _row_spec(8),
            _full_spec((3, d)),
            _full_spec((d, dn)),
        ],
        out_specs=_chunk_spec(Cn),
        out_shape=jax.ShapeDtypeStruct((Cn, NP, F), jnp.float32),
    )(s, hp, dis, bgt, wt)


def _tc_final(s, hp, dis, b7p):
    return pl.pallas_call(
        _final_kernel,
        grid=(NGRID,),
        in_specs=[
            _chunk_spec(2),
            _chunk_spec(1),
            _row_spec(8),
            _full_spec((1, F)),
        ],
        out_specs=_row_spec(F),
        out_shape=jax.ShapeDtypeStruct((NP, F), jnp.float32),
    )(s, hp, dis, b7p)


def _tc_ew(ea2d):
    return pl.pallas_call(
        _ew_kernel,
        grid=(EP // 128 // B,),
        in_specs=[pl.BlockSpec((B, 128), lambda i: (i, 0))],
        out_specs=pl.BlockSpec((B, 128), lambda i: (i, 0)),
        out_shape=jax.ShapeDtypeStruct((EP // 128, 128), jnp.float32),
    )(ea2d)


def kernel(x, edge_index, edge_attr, W1, W2, W3, W4, W5, W6, W7,
           b1, b2, b3, b4, b5, b6, b7, g1, g2, g3, g4, g5, g6,
           t1, t2, t3, t4, t5, t6):
    f32 = jnp.float32
    pad_e = EP - E
    row_pad = jnp.concatenate(
        [edge_index[0], jnp.zeros((pad_e,), jnp.int32)])
    col_pad = jnp.concatenate(
        [edge_index[1], jnp.full((pad_e,), N, jnp.int32)])
    ea_pad = jnp.concatenate([edge_attr[:, 0], jnp.zeros((pad_e,), f32)])
    x_pad = jnp.concatenate([x, jnp.zeros((NP - N, 3), f32)], axis=0)

    ones_in = jnp.ones((B, 8), f32)
    zeros_h = jnp.zeros((RPT, 8), f32)
    zeros_l = jnp.zeros((ZROWS, F), f32)

    row2d = row_pad.reshape(EP // B, B)
    col2d = col_pad.reshape(EP // B, B)
    ew2d = _tc_ew(ea_pad.reshape(EP // 128, 128)).reshape(EP // B, B)
    hist = _hist_call(col2d, ones_in, zeros_h)

    w1t = W1.T
    hp, dis = _tc_pre(x_pad, hist, w1t)

    lk4 = _make_layer_kernel(4)
    lk2 = _make_layer_kernel(2)
    lk1es = _make_layer_kernel(1, edge_split=True)

    Ws = [W2, W3, W4, W5, W6]
    bgts = [jnp.stack([b, g, t]) for b, g, t in
            [(b1, g1, t1), (b2, g2, t2), (b3, g3, t3),
             (b4, g4, t4), (b5, g5, t5), (b6, g6, t6)]]
    dims = [128, 128, 128, 64, 64, 64]
    for li in range(6):
        d = dims[li]
        C = d // F
        lk = {4: lk4, 2: lk2}[C]
        s = lk(hp, row2d, col2d, ew2d, zeros_l)
        if li < 5:
            dn = dims[li + 1]
            wt = Ws[li].T
        else:
            dn = F
            wt = jnp.concatenate(
                [W7.T, jnp.zeros((64, F - 1), f32)], axis=1)
        Cn = dn // F
        hp = _tc_mid(C, Cn, d, dn, s, hp, dis, bgts[li], wt)

    s7 = lk1es(hp, row2d, col2d, ew2d, zeros_l)
    b7p = jnp.concatenate([b7, jnp.zeros((F - 1,), f32)]).reshape(1, F)
    out_full = _tc_final(s7, hp, dis, b7p)
    return out_full[:N, 0:1]
